# Initial kernel scaffold; baseline (speedup 1.0000x reference)
import jax
import jax.numpy as jnp
from jax.experimental import pallas as pl


def kernel(indices, table):
    raise NotImplementedError("write your pallas kernel here")


# speedup vs baseline: 3.3401x; 3.3401x over previous
import functools

import jax
import jax.numpy as jnp
from jax import lax
from jax.experimental import pallas as pl
from jax.experimental.pallas import tpu as pltpu
from jax.experimental.pallas import tpu_sc as plsc

NC = 2
NS = 16
NW = NC * NS
CHUNK = 128
NBUF = 2


@functools.lru_cache(maxsize=None)
def _make_gather(n_rows: int, d: int):
    assert n_rows % (NW * CHUNK) == 0
    nchunks = n_rows // (NW * CHUNK)
    assert nchunks % NBUF == 0
    per_w = nchunks * CHUNK

    mesh = plsc.VectorSubcoreMesh(core_axis_name="c", subcore_axis_name="s")

    @functools.partial(
        pl.kernel,
        mesh=mesh,
        out_type=jax.ShapeDtypeStruct((n_rows, d), jnp.float32),
        scratch_types=[
            pltpu.VMEM((nchunks, CHUNK), jnp.int32),
            pltpu.VMEM((NBUF, CHUNK, d), jnp.float32),
            pltpu.SemaphoreType.DMA,
            pltpu.SemaphoreType.DMA,
        ],
    )
    def gather(table_hbm, idx_hbm, out_hbm, idx_v, rows_v, sem0, sem1):
        sems = (sem0, sem1)
        c = lax.axis_index("c")
        s = lax.axis_index("s")
        wid = s * NC + c
        base = wid * per_w

        pltpu.sync_copy(idx_hbm.at[wid], idx_v)

        for b in range(NBUF):
            pltpu.async_copy(table_hbm.at[idx_v.at[b]], rows_v.at[b], sems[b])

        def body(i, carry):
            g = i * NBUF
            for b in range(NBUF):
                j = g + b
                pltpu.make_async_copy(
                    table_hbm.at[idx_v.at[j]], rows_v.at[b], sems[b]
                ).wait()
                pltpu.sync_copy(
                    rows_v.at[b], out_hbm.at[pl.ds(base + j * CHUNK, CHUNK)]
                )

                @pl.when(j + NBUF < nchunks)
                def _():
                    pltpu.async_copy(
                        table_hbm.at[idx_v.at[j + NBUF]], rows_v.at[b], sems[b]
                    )

            return carry

        lax.fori_loop(0, nchunks // NBUF, body, 0)

    return gather


def kernel(indices, table):
    batch, hist = indices.shape
    _, d = table.shape
    n_rows = batch * hist
    idx = indices.reshape(NW, n_rows // (NW * CHUNK), CHUNK)
    out = _make_gather(n_rows, d)(table, idx)
    return out.reshape(batch, hist, d)


# speedup vs baseline: 3.3403x; 1.0000x over previous
import functools

import jax
import jax.numpy as jnp
from jax import lax
from jax.experimental import pallas as pl
from jax.experimental.pallas import tpu as pltpu
from jax.experimental.pallas import tpu_sc as plsc

NC = 2
NS = 16
NW = NC * NS
CHUNK = 128
NBUF = 5


@functools.lru_cache(maxsize=None)
def _make_gather(n_rows: int, d: int):
    assert n_rows % (NW * CHUNK) == 0
    nchunks = n_rows // (NW * CHUNK)
    assert nchunks % NBUF == 0
    per_w = nchunks * CHUNK

    mesh = plsc.VectorSubcoreMesh(core_axis_name="c", subcore_axis_name="s")

    @functools.partial(
        pl.kernel,
        mesh=mesh,
        out_type=jax.ShapeDtypeStruct((n_rows, d), jnp.float32),
        scratch_types=[
            pltpu.VMEM((nchunks, CHUNK), jnp.int32),
            pltpu.VMEM((NBUF, CHUNK, d), jnp.float32),
        ]
        + [pltpu.SemaphoreType.DMA] * NBUF,
    )
    def gather(table_hbm, idx_hbm, out_hbm, idx_v, rows_v, *sems):
        c = lax.axis_index("c")
        s = lax.axis_index("s")
        wid = s * NC + c
        base = wid * per_w

        pltpu.sync_copy(idx_hbm.at[wid], idx_v)

        for b in range(NBUF):
            pltpu.async_copy(table_hbm.at[idx_v.at[b]], rows_v.at[b], sems[b])

        def body(i, carry):
            g = i * NBUF
            for b in range(NBUF):
                j = g + b
                pltpu.make_async_copy(
                    table_hbm.at[idx_v.at[j]], rows_v.at[b], sems[b]
                ).wait()
                pltpu.sync_copy(
                    rows_v.at[b], out_hbm.at[pl.ds(base + j * CHUNK, CHUNK)]
                )

                @pl.when(j + NBUF < nchunks)
                def _():
                    pltpu.async_copy(
                        table_hbm.at[idx_v.at[j + NBUF]], rows_v.at[b], sems[b]
                    )

            return carry

        lax.fori_loop(0, nchunks // NBUF, body, 0)

    return gather


def kernel(indices, table):
    batch, hist = indices.shape
    _, d = table.shape
    n_rows = batch * hist
    idx = indices.reshape(NW, n_rows // (NW * CHUNK), CHUNK)
    out = _make_gather(n_rows, d)(table, idx)
    return out.reshape(batch, hist, d)


# speedup vs baseline: 5.9354x; 1.7769x over previous
import functools

import jax
import jax.numpy as jnp
from jax import lax
from jax.experimental import pallas as pl
from jax.experimental.pallas import tpu as pltpu
from jax.experimental.pallas import tpu_sc as plsc

NC = 2
NS = 16
NW = NC * NS
BPC = 2
NBUF = 4


@functools.lru_cache(maxsize=None)
def _make_gather(batch: int, hist: int, d: int):
    assert batch % NW == 0
    bat_per_w = batch // NW
    assert bat_per_w % (BPC * NBUF) == 0
    nchunks = bat_per_w // BPC
    rows_per_chunk = BPC * hist

    mesh = plsc.VectorSubcoreMesh(core_axis_name="c", subcore_axis_name="s")

    @functools.partial(
        pl.kernel,
        mesh=mesh,
        out_type=jax.ShapeDtypeStruct((batch, hist, d), jnp.float32),
        scratch_types=[
            pltpu.VMEM((nchunks, rows_per_chunk), jnp.int32),
            pltpu.VMEM((NBUF, rows_per_chunk, d), jnp.float32),
        ]
        + [pltpu.SemaphoreType.DMA] * NBUF,
    )
    def gather(table_hbm, idx_hbm, out_hbm, idx_v, rows_v, *sems):
        c = lax.axis_index("c")
        s = lax.axis_index("s")
        wid = s * NC + c
        bat0 = wid * bat_per_w

        pltpu.sync_copy(idx_hbm.at[wid], idx_v)

        for b in range(NBUF):
            pltpu.async_copy(table_hbm.at[idx_v.at[b]], rows_v.at[b], sems[b])

        def body(i, carry):
            g = i * NBUF
            for b in range(NBUF):
                j = g + b
                pltpu.make_async_copy(
                    table_hbm.at[idx_v.at[j]], rows_v.at[b], sems[b]
                ).wait()
                for q in range(BPC):
                    pltpu.sync_copy(
                        rows_v.at[b, pl.ds(q * hist, hist)],
                        out_hbm.at[bat0 + j * BPC + q],
                    )

                @pl.when(j + NBUF < nchunks)
                def _():
                    pltpu.async_copy(
                        table_hbm.at[idx_v.at[j + NBUF]], rows_v.at[b], sems[b]
                    )

            return carry

        lax.fori_loop(0, nchunks // NBUF, body, 0)

    return gather


def kernel(indices, table):
    batch, hist = indices.shape
    _, d = table.shape
    idx = indices.reshape(NW, batch // (NW * BPC), BPC * hist)
    return _make_gather(batch, hist, d)(table, idx)


# speedup vs baseline: 5.9415x; 1.0010x over previous
import functools

import jax
import jax.numpy as jnp
from jax import lax
from jax.experimental import pallas as pl
from jax.experimental.pallas import tpu as pltpu
from jax.experimental.pallas import tpu_sc as plsc

NC = 2
NS = 16
NW = NC * NS
BPC = 2
NBUF = 4


@functools.lru_cache(maxsize=None)
def _make_gather(batch: int, hist: int, d: int):
    assert batch % NW == 0
    bat_per_w = batch // NW
    assert bat_per_w % (BPC * NBUF) == 0
    nchunks = bat_per_w // BPC
    rows_per_chunk = BPC * hist

    mesh = plsc.VectorSubcoreMesh(core_axis_name="c", subcore_axis_name="s")

    @functools.partial(
        pl.kernel,
        mesh=mesh,
        out_type=jax.ShapeDtypeStruct((batch, hist, d), jnp.float32),
        scratch_types=[
            pltpu.VMEM((nchunks, 128), jnp.int32),
            pltpu.VMEM((NBUF, rows_per_chunk, d), jnp.float32),
        ]
        + [pltpu.SemaphoreType.DMA] * NBUF,
        compiler_params=pltpu.CompilerParams(use_tc_tiling_on_sc=True),
    )
    def gather(table_hbm, idx_hbm, out_hbm, idx_v, rows_v, *sems):
        c = lax.axis_index("c")
        s = lax.axis_index("s")
        wid = s * NC + c
        bat0 = wid * bat_per_w

        pltpu.sync_copy(idx_hbm.at[wid], idx_v)

        for b in range(NBUF):
            pltpu.async_copy(
                table_hbm.at[idx_v.at[b, pl.ds(0, rows_per_chunk)]],
                rows_v.at[b],
                sems[b],
            )

        def body(i, carry):
            g = i * NBUF
            for b in range(NBUF):
                j = g + b
                pltpu.make_async_copy(
                    table_hbm.at[idx_v.at[j, pl.ds(0, rows_per_chunk)]],
                    rows_v.at[b],
                    sems[b],
                ).wait()
                for q in range(BPC):
                    pltpu.sync_copy(
                        rows_v.at[b, pl.ds(q * hist, hist)],
                        out_hbm.at[bat0 + j * BPC + q],
                    )

                @pl.when(j + NBUF < nchunks)
                def _():
                    pltpu.async_copy(
                        table_hbm.at[idx_v.at[j + NBUF, pl.ds(0, rows_per_chunk)]],
                        rows_v.at[b],
                        sems[b],
                    )

            return carry

        lax.fori_loop(0, nchunks // NBUF, body, 0)

    return gather


def kernel(indices, table):
    batch, hist = indices.shape
    _, d = table.shape
    rows_per_chunk = BPC * hist
    idx = indices.reshape(NW, batch // (NW * BPC), rows_per_chunk)
    idx = jnp.pad(idx, ((0, 0), (0, 0), (0, 128 - rows_per_chunk)))
    return _make_gather(batch, hist, d)(table, idx)


# speedup vs baseline: 10.7185x; 1.8040x over previous
import functools

import jax
import jax.numpy as jnp
from jax import lax
from jax.experimental import pallas as pl
from jax.experimental.pallas import tpu as pltpu
from jax.experimental.pallas import tpu_sc as plsc

NC = 2
NS = 16
NW = NC * NS
NBUF = 5


@functools.lru_cache(maxsize=None)
def _make_gather(batch: int, hist: int, d: int):
    assert batch % NW == 0
    bat_per_w = batch // NW
    assert bat_per_w <= 128
    assert hist % NBUF == 0

    mesh = plsc.VectorSubcoreMesh(core_axis_name="c", subcore_axis_name="s")

    @functools.partial(
        pl.kernel,
        mesh=mesh,
        out_type=jax.ShapeDtypeStruct((hist, batch, d), jnp.float32),
        scratch_types=[
            pltpu.VMEM((hist, bat_per_w), jnp.int32),
            pltpu.VMEM((NBUF, bat_per_w, d), jnp.float32),
        ]
        + [pltpu.SemaphoreType.DMA] * NBUF,
    )
    def gather(table_hbm, idxt_hbm, out_hbm, idx_v, rows_v, *sems):
        c = lax.axis_index("c")
        s = lax.axis_index("s")
        wid = s * NC + c
        bat0 = wid * bat_per_w

        pltpu.sync_copy(idxt_hbm.at[:, wid], idx_v)

        for b in range(NBUF):
            pltpu.async_copy(table_hbm.at[idx_v.at[b]], rows_v.at[b], sems[b])

        def body(i, carry):
            g = i * NBUF
            for b in range(NBUF):
                j = g + b
                pltpu.make_async_copy(
                    table_hbm.at[idx_v.at[j]], rows_v.at[b], sems[b]
                ).wait()
                pltpu.sync_copy(
                    rows_v.at[b], out_hbm.at[j, pl.ds(bat0, bat_per_w)]
                )

                @pl.when(j + NBUF < hist)
                def _():
                    pltpu.async_copy(
                        table_hbm.at[idx_v.at[j + NBUF]], rows_v.at[b], sems[b]
                    )

            return carry

        lax.fori_loop(0, hist // NBUF, body, 0)

    return gather


def kernel(indices, table):
    batch, hist = indices.shape
    _, d = table.shape
    idxt = jnp.transpose(indices).reshape(hist, NW, batch // NW)
    outt = _make_gather(batch, hist, d)(table, idxt)
    return jnp.transpose(outt, (1, 0, 2))
